# baseline (speedup 1.0000x reference)
import dataclasses
import functools

import jax
import jax.numpy as jnp
from jax import lax
from jax.experimental import pallas as pl
from jax.experimental.pallas import tpu as pltpu
from jax.experimental.pallas import tpu_sc as plsc

_VPAD = 128
_NC, _NS = 2, 16
_NW = _NC * _NS
_LANES = 16
_CHUNK = 256
_LP = 64


@jax.jit
def _sc_call(idx, scores, w1, w2, prior, bias, thresh):
    nrow, ncol = idx.shape
    rows_w = nrow // _NW
    nch = rows_w // _CHUNK
    mesh = plsc.VectorSubcoreMesh(core_axis_name="c", subcore_axis_name="s")
    cp = pltpu.CompilerParams()
    if "needs_layout_passes" in pltpu.CompilerParams.__dataclass_fields__:
        cp = dataclasses.replace(cp, needs_layout_passes=False)
    cp = dataclasses.replace(cp, use_tc_tiling_on_sc=False)

    @functools.partial(
        pl.kernel,
        out_type=[
            jax.ShapeDtypeStruct((nrow, ncol), jnp.float32),
            jax.ShapeDtypeStruct((nrow, ncol), jnp.int32),
        ],
        mesh=mesh,
        scratch_types=[
            pltpu.VMEM((_CHUNK, 50), jnp.int32),
            pltpu.VMEM((_CHUNK, 50), jnp.float32),
            pltpu.VMEM((_CHUNK, 50), jnp.float32),
            pltpu.VMEM((_CHUNK, 50), jnp.int32),
            pltpu.VMEM((_VPAD,), jnp.float32),
            pltpu.VMEM((_VPAD,), jnp.float32),
            pltpu.VMEM((_VPAD,), jnp.float32),
            pltpu.VMEM((_VPAD,), jnp.float32),
            pltpu.VMEM((_VPAD,), jnp.float32),
        ],
        compiler_params=cp,
    )
    def body(idx_hbm, s_hbm, w1_hbm, w2_hbm, pr_hbm, bi_hbm, th_hbm,
             cal_hbm, mask_hbm,
             idx_v, s_v, cal_v, m_v, w1_v, c_v, pr_v, bi_v, th_v):
        wid = lax.axis_index("s") * _NC + lax.axis_index("c")
        row0 = wid * rows_w
        pltpu.sync_copy(w1_hbm, w1_v)
        pltpu.sync_copy(w2_hbm, c_v)
        pltpu.sync_copy(pr_hbm, pr_v)
        pltpu.sync_copy(bi_hbm, bi_v)
        pltpu.sync_copy(th_hbm, th_v)

        @pl.loop(0, _VPAD, step=_LANES)
        def _(i):
            sl = pl.ds(i, _LANES)
            c_v[sl] = -(c_v[sl] * pr_v[sl] + bi_v[sl])
            w1_v[sl] = -w1_v[sl]

        col_starts = list(range(0, ncol - _LANES, _LANES)) + [ncol - _LANES]

        @pl.loop(0, nch)
        def _(ch):
            rsl = pl.ds(row0 + ch * _CHUNK, _CHUNK)
            pltpu.sync_copy(idx_hbm.at[rsl, :], idx_v)
            pltpu.sync_copy(s_hbm.at[rsl, :], s_v)

            @plsc.parallel_loop(0, _CHUNK, unroll=4)
            def _(r):
                for c in col_starts:
                    sl = pl.ds(c, _LANES)
                    ids = idx_v[r, sl]
                    na = plsc.load_gather(w1_v, [ids])
                    nc2 = plsc.load_gather(c_v, [ids])
                    th = plsc.load_gather(th_v, [ids])
                    e = jnp.exp(na * s_v[r, sl] + nc2)
                    cal = 1.0 / (1.0 + e)
                    cal_v[r, sl] = cal
                    m_v[r, sl] = jnp.where(cal > th, jnp.int32(1), jnp.int32(0))

            pltpu.sync_copy(cal_v, cal_hbm.at[rsl, :])
            pltpu.sync_copy(m_v, mask_hbm.at[rsl, :])

    return body(idx, scores, w1, w2, prior, bias, thresh)


def kernel(type_ids, scores, prior, weights, bias, threshold):
    v = prior.shape[0]
    pad = _VPAD - v
    idx = type_ids.astype(jnp.int32)
    w1 = jnp.pad(weights[:, 0], (0, pad))
    w2 = jnp.pad(weights[:, 1], (0, pad))
    pr = jnp.pad(prior, (0, pad))
    bi = jnp.pad(bias, (0, pad))
    th = jnp.pad(threshold, (0, pad))
    cal, mask = _sc_call(idx, scores, w1, w2, pr, bi, th)
    return cal, mask.astype(jnp.bool_)


# speedup vs baseline: 1.4272x; 1.4272x over previous
import dataclasses
import functools

import jax
import jax.numpy as jnp
from jax import lax
from jax.experimental import pallas as pl
from jax.experimental.pallas import tpu as pltpu
from jax.experimental.pallas import tpu_sc as plsc

_VPAD = 128
_NC, _NS = 2, 16
_NW = _NC * _NS
_LANES = 16
_CHUNK = 256
_CSL = 56


@functools.partial(jax.jit, static_argnames=("ncol",))
def _sc_call(idx, scores, w1, w2, prior, bias, thresh, *, ncol):
    nrow = idx.shape[0]
    rows_w = nrow // _NW
    nch = rows_w // _CHUNK
    mesh = plsc.VectorSubcoreMesh(core_axis_name="c", subcore_axis_name="s")
    cp = pltpu.CompilerParams()
    if "needs_layout_passes" in pltpu.CompilerParams.__dataclass_fields__:
        cp = dataclasses.replace(cp, needs_layout_passes=False)
    cp = dataclasses.replace(cp, use_tc_tiling_on_sc=False)

    @functools.partial(
        pl.kernel,
        out_type=[
            jax.ShapeDtypeStruct((nrow, 128), jnp.float32),
            jax.ShapeDtypeStruct((nrow, 128), jnp.int32),
        ],
        mesh=mesh,
        scratch_types=[
            pltpu.VMEM((_CHUNK, _CSL), jnp.int32),
            pltpu.VMEM((_CHUNK, _CSL), jnp.float32),
            pltpu.VMEM((_CHUNK, _CSL), jnp.float32),
            pltpu.VMEM((_CHUNK, _CSL), jnp.int32),
            pltpu.VMEM((_VPAD,), jnp.float32),
            pltpu.VMEM((_VPAD,), jnp.float32),
            pltpu.VMEM((_VPAD,), jnp.float32),
            pltpu.VMEM((_VPAD,), jnp.float32),
            pltpu.VMEM((_VPAD,), jnp.float32),
        ],
        compiler_params=cp,
    )
    def body(idx_hbm, s_hbm, w1_hbm, w2_hbm, pr_hbm, bi_hbm, th_hbm,
             cal_hbm, mask_hbm,
             idx_v, s_v, cal_v, m_v, w1_v, c_v, pr_v, bi_v, th_v):
        wid = lax.axis_index("s") * _NC + lax.axis_index("c")
        row0 = wid * rows_w
        pltpu.sync_copy(w1_hbm, w1_v)
        pltpu.sync_copy(w2_hbm, c_v)
        pltpu.sync_copy(pr_hbm, pr_v)
        pltpu.sync_copy(bi_hbm, bi_v)
        pltpu.sync_copy(th_hbm, th_v)

        @pl.loop(0, _VPAD, step=_LANES)
        def _(i):
            sl = pl.ds(i, _LANES)
            c_v[sl] = -(c_v[sl] * pr_v[sl] + bi_v[sl])
            w1_v[sl] = -w1_v[sl]

        col_starts = list(range(0, ncol - _LANES, _LANES)) + [ncol - _LANES]
        csl = pl.ds(0, _CSL)

        @pl.loop(0, nch)
        def _(ch):
            rsl = pl.ds(row0 + ch * _CHUNK, _CHUNK)
            pltpu.sync_copy(idx_hbm.at[rsl, csl], idx_v)
            pltpu.sync_copy(s_hbm.at[rsl, csl], s_v)

            @plsc.parallel_loop(0, _CHUNK, unroll=4)
            def _(r):
                for c in col_starts:
                    sl = pl.ds(c, _LANES)
                    ids = idx_v[r, sl]
                    na = plsc.load_gather(w1_v, [ids])
                    nc2 = plsc.load_gather(c_v, [ids])
                    th = plsc.load_gather(th_v, [ids])
                    e = jnp.exp(na * s_v[r, sl] + nc2)
                    cal = 1.0 / (1.0 + e)
                    cal_v[r, sl] = cal
                    m_v[r, sl] = jnp.where(cal > th, jnp.int32(1), jnp.int32(0))

            pltpu.sync_copy(cal_v, cal_hbm.at[rsl, csl])
            pltpu.sync_copy(m_v, mask_hbm.at[rsl, csl])

    return body(idx, scores, w1, w2, prior, bias, thresh)


def kernel(type_ids, scores, prior, weights, bias, threshold):
    v = prior.shape[0]
    pad = _VPAD - v
    ncol = type_ids.shape[1]
    cpad = ((0, 0), (0, 128 - ncol))
    idx = jnp.pad(type_ids.astype(jnp.int32), cpad)
    s = jnp.pad(scores, cpad)
    w1 = jnp.pad(weights[:, 0], (0, pad))
    w2 = jnp.pad(weights[:, 1], (0, pad))
    pr = jnp.pad(prior, (0, pad))
    bi = jnp.pad(bias, (0, pad))
    th = jnp.pad(threshold, (0, pad))
    cal, mask = _sc_call(idx, s, w1, w2, pr, bi, th, ncol=ncol)
    return cal[:, :ncol], mask[:, :ncol].astype(jnp.bool_)
